# Initial kernel scaffold; baseline (speedup 1.0000x reference)
import jax
import jax.numpy as jnp
from jax.experimental import pallas as pl


def kernel(x_user, x_recipe, x_ing, params, edge_rates, edge_rev_rates, edge_has, edge_rev_has):
    raise NotImplementedError("write your pallas kernel here")


# speedup vs baseline: 2.0785x; 2.0785x over previous
import functools

import jax
import jax.numpy as jnp
from jax import lax
from jax.experimental import pallas as pl
from jax.experimental.pallas import tpu as pltpu
from jax.experimental.pallas import tpu_sc as plsc

NC = 2
NS = 16
NW = NC * NS
K = 128
CW = 32
ZR = 125


def _sc_segment_sums(tables_by_type, edges, n_dsts, with_counts):
    ntypes = len(edges)
    E = edges[0].shape[1]
    nblk = E // K
    nj = (nblk + NW - 1) // NW
    nchs = [len(t) for t in tables_by_type]

    out_types = []
    for t in range(ntypes):
        cch = nchs[t] + (1 if with_counts else 0)
        out_types.append(
            jax.ShapeDtypeStruct((NC, cch, n_dsts[t], CW), jnp.float32))

    mesh = plsc.VectorSubcoreMesh(core_axis_name="c", subcore_axis_name="s")

    @functools.partial(
        pl.kernel,
        out_type=tuple(out_types),
        mesh=mesh,
        compiler_params=pltpu.CompilerParams(use_tc_tiling_on_sc=False),
        scratch_types=[
            pltpu.VMEM((K,), jnp.int32),
            pltpu.VMEM((K,), jnp.int32),
            pltpu.VMEM((K, CW), jnp.float32),
            pltpu.VMEM((K, CW), jnp.float32),
            pltpu.VMEM((ZR, CW), jnp.float32),
            pltpu.VMEM_SHARED((50000, CW), jnp.float32),
            pltpu.SemaphoreType.DMA,
        ],
    )
    def sc_kernel(*refs):
        nin = sum(nchs) + ntypes
        table_refs = []
        pos = 0
        flat_tables = refs[:sum(nchs)]
        for t in range(ntypes):
            table_refs.append(flat_tables[pos:pos + nchs[t]])
            pos += nchs[t]
        edge_refs = refs[sum(nchs):nin]
        out_refs = refs[nin:nin + ntypes]
        idx_s, idx_d, rows, ones, zbuf, acc, sem = refs[nin + ntypes:]

        cid = lax.axis_index("c")
        sid = lax.axis_index("s")
        wid = sid * NC + cid

        def init_rows(i, _):
            for h in range(CW // 16):
                zbuf[i, pl.ds(16 * h, 16)] = jnp.zeros((16,), jnp.float32)
            return 0
        lax.fori_loop(0, ZR, init_rows, 0)
        if with_counts:
            def init_ones(i, _):
                for h in range(CW // 16):
                    ones[i, pl.ds(16 * h, 16)] = jnp.ones((16,), jnp.float32)
                return 0
            lax.fori_loop(0, K, init_ones, 0)

        def run_pass(edge_ref, table_ref, out_ref, ck, n_dst):
            rpt = n_dst // NS
            nz = rpt // ZR
            base = sid * rpt

            def zb(i, _):
                pltpu.sync_copy(zbuf, acc.at[pl.ds(base + i * ZR, ZR)])
                return 0
            lax.fori_loop(0, nz, zb, 0)
            plsc.subcore_barrier()

            def eb(j, _):
                bid = wid + j * NW
                @pl.when(bid < nblk)
                def _():
                    pltpu.sync_copy(edge_ref.at[1, pl.ds(bid * K, K)], idx_d)
                    if table_ref is not None:
                        pltpu.sync_copy(
                            edge_ref.at[0, pl.ds(bid * K, K)], idx_s)
                        pltpu.async_copy(
                            table_ref.at[idx_s], rows, sem).wait()
                        pltpu.sync_copy(rows, acc.at[idx_d], add=True)
                    else:
                        pltpu.sync_copy(ones, acc.at[idx_d], add=True)
                return 0
            lax.fori_loop(0, nj, eb, 0)
            plsc.subcore_barrier()

            def db(i, _):
                sl = pl.ds(base + i * ZR, ZR)
                pltpu.sync_copy(acc.at[sl], out_ref.at[cid, ck, sl])
                return 0
            lax.fori_loop(0, nz, db, 0)

        for t in range(ntypes):
            for c in range(nchs[t]):
                run_pass(edge_refs[t], table_refs[t][c], out_refs[t], c,
                         n_dsts[t])
            if with_counts:
                run_pass(edge_refs[t], None, out_refs[t], nchs[t],
                         n_dsts[t])

    flat_in = [tb for t in tables_by_type for tb in t] + list(edges)
    return sc_kernel(*flat_in)


def _dot_t(a, w):
    return lax.dot_general(
        a, w, (((1,), (1,)), ((), ())),
        precision=lax.Precision.HIGHEST,
        preferred_element_type=jnp.float32)


def _tc_layer1(n, d_in, Ss, x, Ws, B=2000):
    R = len(Ss)
    nch = d_in // CW
    B8 = B // 8

    def body(*refs):
        S = refs[:R]
        x_ref = refs[R]
        W = refs[R + 1:R + 1 + 3 * R]
        out = refs[R + 1 + 3 * R:]
        h_ref = out[0]
        hc_refs = out[1:5]
        rc_refs = out[5:5 + R]

        h = None
        for r in range(R):
            Sr = S[r]
            cnt = (Sr[0, nch] + Sr[1, nch]).reshape(B, CW)
            rc = 1.0 / jnp.maximum(cnt, 1.0)
            rc_refs[r][...] = rc.reshape(B8, 8, CW)
            Wl = W[3 * r][...]
            o = jnp.broadcast_to(W[3 * r + 1][...], (B, 128))
            for c in range(nch):
                mean_c = (Sr[0, c] + Sr[1, c]).reshape(B, CW) * rc
                o = o + _dot_t(mean_c, Wl[:, CW * c:CW * (c + 1)])
            o = o + _dot_t(x_ref[...], W[3 * r + 2][...])
            h = o if h is None else h + o
        if R > 1:
            h = h / float(R)
        h = jnp.maximum(h, 0.0)
        h_ref[...] = h
        for c in range(4):
            hc_refs[c][...] = h[:, CW * c:CW * (c + 1)].reshape(B8, 8, CW)

    grid = (n // B,)
    in_specs = (
        [pl.BlockSpec((NC, nch + 1, B8, 8, CW),
                      lambda i: (0, 0, i, 0, 0)) for _ in range(R)]
        + [pl.BlockSpec((B, d_in), lambda i: (i, 0))]
        + [spec for _ in range(R) for spec in (
            pl.BlockSpec((128, d_in), lambda i: (0, 0)),
            pl.BlockSpec((1, 128), lambda i: (0, 0)),
            pl.BlockSpec((128, d_in), lambda i: (0, 0)))]
    )
    out_shapes = ([jax.ShapeDtypeStruct((n, 128), jnp.float32)]
                  + [jax.ShapeDtypeStruct((n // 8, 8, CW), jnp.float32)
                     for _ in range(4 + R)])
    out_specs = ([pl.BlockSpec((B, 128), lambda i: (i, 0))]
                 + [pl.BlockSpec((B8, 8, CW), lambda i: (i, 0, 0))
                    for _ in range(4 + R)])
    args = list(Ss) + [x] + [w for ws in Ws for w in ws]
    outs = pl.pallas_call(
        body, grid=grid, in_specs=in_specs, out_specs=out_specs,
        out_shape=out_shapes)(*args)
    return outs[0], list(outs[1:5]), list(outs[5:5 + R])


def _tc_layer2(n, Ss, rcs, h1, Ws, B=2000):
    R = len(Ss)
    nch = 128 // CW
    B8 = B // 8

    def body(*refs):
        S = refs[:R]
        rc_refs = refs[R:2 * R]
        h1_ref = refs[2 * R]
        W = refs[2 * R + 1:2 * R + 1 + 3 * R]
        out_ref = refs[-1]

        h1v = h1_ref[...]
        h = None
        for r in range(R):
            Sr = S[r]
            rc = rc_refs[r][...].reshape(B, CW)
            Wl = W[3 * r][...]
            o = jnp.broadcast_to(W[3 * r + 1][...], (B, 128))
            for c in range(nch):
                mean_c = (Sr[0, c] + Sr[1, c]).reshape(B, CW) * rc
                o = o + _dot_t(mean_c, Wl[:, CW * c:CW * (c + 1)])
            o = o + _dot_t(h1v, W[3 * r + 2][...])
            h = o if h is None else h + o
        if R > 1:
            h = h / float(R)
        out_ref[...] = (h1v + h) * 0.5

    grid = (n // B,)
    in_specs = (
        [pl.BlockSpec((NC, nch, B8, 8, CW), lambda i: (0, 0, i, 0, 0))
         for _ in range(R)]
        + [pl.BlockSpec((B8, 8, CW), lambda i: (i, 0, 0))
           for _ in range(R)]
        + [pl.BlockSpec((B, 128), lambda i: (i, 0))]
        + [spec for _ in range(R) for spec in (
            pl.BlockSpec((128, 128), lambda i: (0, 0)),
            pl.BlockSpec((1, 128), lambda i: (0, 0)),
            pl.BlockSpec((128, 128), lambda i: (0, 0)))]
    )
    args = list(Ss) + list(rcs) + [h1] + [w for ws in Ws for w in ws]
    return pl.pallas_call(
        body, grid=grid, in_specs=in_specs,
        out_specs=pl.BlockSpec((B, 128), lambda i: (i, 0)),
        out_shape=jax.ShapeDtypeStruct((n, 128), jnp.float32))(*args)


def kernel(x_user, x_recipe, x_ing, params, edge_rates, edge_rev_rates,
           edge_has, edge_rev_has):
    p = params
    n_user, d_in = x_user.shape
    n_recipe = x_recipe.shape[0]
    n_ing = x_ing.shape[0]
    nch1 = d_in // CW

    def chunks(x):
        return [x[:, CW * c:CW * (c + 1)] for c in range(x.shape[1] // CW)]

    def wset(pref, et):
        return (p[pref + "_" + et + "_Wl"],
                p[pref + "_" + et + "_b"].reshape(1, 128),
                p[pref + "_" + et + "_Wr"])

    xu_c, xr_c, xi_c = chunks(x_user), chunks(x_recipe), chunks(x_ing)

    S1 = _sc_segment_sums(
        [xu_c, xr_c, xr_c, xi_c],
        [edge_rates, edge_rev_rates, edge_has, edge_rev_has],
        [n_recipe, n_user, n_ing, n_recipe],
        with_counts=True)
    S1_ra, S1_rr, S1_ha, S1_rh = [
        s.reshape(NC, nch1 + 1, s.shape[2] // 8, 8, CW) for s in S1]

    h1r, h1r_c, (rc_ra, rc_rh) = _tc_layer1(
        n_recipe, d_in, [S1_ra, S1_rh], x_recipe,
        [wset("l1", "rates"), wset("l1", "rev_has")])
    h1u, h1u_c, (rc_rr,) = _tc_layer1(
        n_user, d_in, [S1_rr], x_user, [wset("l1", "rev_rates")])
    h1i, h1i_c, (rc_ha,) = _tc_layer1(
        n_ing, d_in, [S1_ha], x_ing, [wset("l1", "has")])

    def unfold(cs):
        return [c.reshape(-1, CW) for c in cs]

    S2 = _sc_segment_sums(
        [unfold(h1u_c), unfold(h1r_c), unfold(h1r_c), unfold(h1i_c)],
        [edge_rates, edge_rev_rates, edge_has, edge_rev_has],
        [n_recipe, n_user, n_ing, n_recipe],
        with_counts=False)
    S2_ra, S2_rr, S2_ha, S2_rh = [
        s.reshape(NC, 4, s.shape[2] // 8, 8, CW) for s in S2]

    hr = _tc_layer2(n_recipe, [S2_ra, S2_rh], [rc_ra, rc_rh], h1r,
                    [wset("l2", "rates"), wset("l2", "rev_has")])
    hu = _tc_layer2(n_user, [S2_rr], [rc_rr], h1u,
                    [wset("l2", "rev_rates")])
    hi = _tc_layer2(n_ing, [S2_ha], [rc_ha], h1i, [wset("l2", "has")])
    return hu, hr, hi


# speedup vs baseline: 3.5457x; 1.7059x over previous
import functools

import jax
import jax.numpy as jnp
from jax import lax
from jax.experimental import pallas as pl
from jax.experimental.pallas import tpu as pltpu
from jax.experimental.pallas import tpu_sc as plsc

NC = 2
NS = 16
NW = NC * NS
K = 128
CW = 32
ZR = 125


def _sc_segment_sums(tables_by_type, edges, n_dsts, with_counts):
    ntypes = len(edges)
    E = edges[0].shape[1]
    nj = -(-E // (NW * K))
    ep = NW * K * nj
    nchs = [len(t) for t in tables_by_type]

    srcs, dsts = [], []
    for ei, nd in zip(edges, n_dsts):
        pad = ep - E
        srcs.append(jnp.concatenate(
            [ei[0], jnp.zeros((pad,), jnp.int32)]).reshape(ep // K, K))
        dsts.append(jnp.concatenate(
            [ei[1], jnp.full((pad,), nd, jnp.int32)]).reshape(ep // K, K))

    out_types = []
    for t in range(ntypes):
        cch = nchs[t] + (1 if with_counts else 0)
        out_types.append(
            jax.ShapeDtypeStruct((NC, cch, n_dsts[t], CW), jnp.float32))

    mesh = plsc.VectorSubcoreMesh(core_axis_name="c", subcore_axis_name="s")

    WB = 14
    nwin = nj // WB
    assert nj == WB * nwin and nwin % 2 == 0

    @functools.partial(
        pl.kernel,
        out_type=tuple(out_types),
        mesh=mesh,
        compiler_params=pltpu.CompilerParams(use_tc_tiling_on_sc=False),
        scratch_types=[
            pltpu.VMEM((WB, K), jnp.int32),
            pltpu.VMEM((WB, K), jnp.int32),
            pltpu.VMEM((WB, K), jnp.int32),
            pltpu.VMEM((WB, K), jnp.int32),
            pltpu.VMEM((K, CW), jnp.float32),
            pltpu.VMEM((K, CW), jnp.float32),
            pltpu.VMEM((K, CW), jnp.float32),
            pltpu.VMEM((ZR, CW), jnp.float32),
            pltpu.VMEM_SHARED((50016, CW), jnp.float32),
            pltpu.SemaphoreType.DMA,
            pltpu.SemaphoreType.DMA,
            pltpu.SemaphoreType.DMA,
        ],
    )
    def sc_kernel(*refs):
        nt = sum(nchs)
        nin = nt + 2 * ntypes
        table_refs = []
        pos = 0
        for t in range(ntypes):
            table_refs.append(refs[pos:pos + nchs[t]])
            pos += nchs[t]
        src_refs = refs[nt:nt + ntypes]
        dst_refs = refs[nt + ntypes:nin]
        out_refs = refs[nin:nin + ntypes]
        (wsA, wdA, wsB, wdB, rows0, rows1, ones, zbuf, acc,
         gsem, ssem, wsem) = refs[nin + ntypes:]
        rows = (rows0, rows1)

        cid = lax.axis_index("c")
        sid = lax.axis_index("s")
        wid = sid * NC + cid

        def init_rows(i, _):
            for h in range(CW // 16):
                zbuf[i, pl.ds(16 * h, 16)] = jnp.zeros((16,), jnp.float32)
            return 0
        lax.fori_loop(0, ZR, init_rows, 0)
        if with_counts:
            def init_ones(i, _):
                for h in range(CW // 16):
                    ones[i, pl.ds(16 * h, 16)] = jnp.ones((16,), jnp.float32)
                return 0
            lax.fori_loop(0, K, init_ones, 0)

        def win_slice(ref_hbm, w):
            return ref_hbm.at[pl.ds(wid * nj + w * WB, WB)]

        def run_pass(t, table_ref, out_ref, ck, n_dst):
            rpt = n_dst // NS
            nz = rpt // ZR
            base = sid * rpt

            def zb(i, _):
                pltpu.sync_copy(zbuf, acc.at[pl.ds(base + i * ZR, ZR)])
                return 0
            lax.fori_loop(0, nz, zb, 0)
            plsc.subcore_barrier()

            if table_ref is not None:
                def process(ws, wd):
                    pltpu.async_copy(table_ref.at[ws.at[0]], rows0, gsem)
                    for b in range(WB):
                        pltpu.make_async_copy(
                            table_ref.at[ws.at[b]], rows[b % 2],
                            gsem).wait()
                        if b + 1 < WB:
                            pltpu.async_copy(
                                table_ref.at[ws.at[b + 1]],
                                rows[(b + 1) % 2], gsem)
                        pltpu.sync_copy(
                            rows[b % 2], acc.at[wd.at[b]], add=True)
            else:
                def process(ws, wd):
                    for h in range(2):
                        for b in range(WB // 2):
                            pltpu.async_copy(
                                ones, acc.at[wd.at[h * (WB // 2) + b]],
                                ssem, add=True)
                        for b in range(WB // 2):
                            pltpu.make_async_copy(
                                ones, acc.at[wd.at[h * (WB // 2) + b]],
                                ssem).wait()

            def start_win(w, ws, wd):
                pltpu.async_copy(win_slice(src_refs[t], w), ws, wsem)
                pltpu.async_copy(win_slice(dst_refs[t], w), wd, wsem)

            def wait_win(w, ws, wd):
                pltpu.make_async_copy(
                    win_slice(src_refs[t], w), ws, wsem).wait()
                pltpu.make_async_copy(
                    win_slice(dst_refs[t], w), wd, wsem).wait()

            pltpu.sync_copy(win_slice(src_refs[t], 0), wsA)
            pltpu.sync_copy(win_slice(dst_refs[t], 0), wdA)
            start_win(1, wsB, wdB)

            def ww_body(ww, _):
                w0 = 2 * ww
                @pl.when(ww > 0)
                def _():
                    wait_win(w0, wsA, wdA)
                process(wsA, wdA)
                wait_win(w0 + 1, wsB, wdB)
                @pl.when(w0 + 2 < nwin)
                def _():
                    start_win(w0 + 2, wsA, wdA)
                process(wsB, wdB)
                @pl.when(w0 + 3 < nwin)
                def _():
                    start_win(w0 + 3, wsB, wdB)
                return 0
            lax.fori_loop(0, nwin // 2, ww_body, 0)
            plsc.subcore_barrier()

            def db(i, _):
                sl = pl.ds(base + i * ZR, ZR)
                pltpu.sync_copy(acc.at[sl], out_ref.at[cid, ck, sl])
                return 0
            lax.fori_loop(0, nz, db, 0)

        for t in range(ntypes):
            for c in range(nchs[t]):
                run_pass(t, table_refs[t][c], out_refs[t], c, n_dsts[t])
            if with_counts:
                run_pass(t, None, out_refs[t], nchs[t], n_dsts[t])

    flat_in = ([tb for t in tables_by_type for tb in t] + srcs + dsts)
    return sc_kernel(*flat_in)


def _dot_t(a, w):
    return lax.dot_general(
        a, w, (((1,), (1,)), ((), ())),
        precision=lax.Precision.HIGHEST,
        preferred_element_type=jnp.float32)


def _tc_layer1(n, d_in, Ss, x, Ws, B=2000):
    R = len(Ss)
    nch = d_in // CW
    B8 = B // 8

    def body(*refs):
        S = refs[:R]
        x_ref = refs[R]
        W = refs[R + 1:R + 1 + 3 * R]
        out = refs[R + 1 + 3 * R:]
        h_ref = out[0]
        hc_refs = out[1:5]
        rc_refs = out[5:5 + R]

        h = None
        for r in range(R):
            Sr = S[r]
            cnt = (Sr[0, nch] + Sr[1, nch]).reshape(B, CW)
            rc = 1.0 / jnp.maximum(cnt, 1.0)
            rc_refs[r][...] = rc.reshape(B8, 8, CW)
            Wl = W[3 * r][...]
            o = jnp.broadcast_to(W[3 * r + 1][...], (B, 128))
            for c in range(nch):
                mean_c = (Sr[0, c] + Sr[1, c]).reshape(B, CW) * rc
                o = o + _dot_t(mean_c, Wl[:, CW * c:CW * (c + 1)])
            o = o + _dot_t(x_ref[...], W[3 * r + 2][...])
            h = o if h is None else h + o
        if R > 1:
            h = h / float(R)
        h = jnp.maximum(h, 0.0)
        h_ref[...] = h
        for c in range(4):
            hc_refs[c][...] = h[:, CW * c:CW * (c + 1)].reshape(B8, 8, CW)

    grid = (n // B,)
    in_specs = (
        [pl.BlockSpec((NC, nch + 1, B8, 8, CW),
                      lambda i: (0, 0, i, 0, 0)) for _ in range(R)]
        + [pl.BlockSpec((B, d_in), lambda i: (i, 0))]
        + [spec for _ in range(R) for spec in (
            pl.BlockSpec((128, d_in), lambda i: (0, 0)),
            pl.BlockSpec((1, 128), lambda i: (0, 0)),
            pl.BlockSpec((128, d_in), lambda i: (0, 0)))]
    )
    out_shapes = ([jax.ShapeDtypeStruct((n, 128), jnp.float32)]
                  + [jax.ShapeDtypeStruct((n // 8, 8, CW), jnp.float32)
                     for _ in range(4 + R)])
    out_specs = ([pl.BlockSpec((B, 128), lambda i: (i, 0))]
                 + [pl.BlockSpec((B8, 8, CW), lambda i: (i, 0, 0))
                    for _ in range(4 + R)])
    args = list(Ss) + [x] + [w for ws in Ws for w in ws]
    outs = pl.pallas_call(
        body, grid=grid, in_specs=in_specs, out_specs=out_specs,
        out_shape=out_shapes)(*args)
    return outs[0], list(outs[1:5]), list(outs[5:5 + R])


def _tc_layer2(n, Ss, rcs, h1, Ws, B=2000):
    R = len(Ss)
    nch = 128 // CW
    B8 = B // 8

    def body(*refs):
        S = refs[:R]
        rc_refs = refs[R:2 * R]
        h1_ref = refs[2 * R]
        W = refs[2 * R + 1:2 * R + 1 + 3 * R]
        out_ref = refs[-1]

        h1v = h1_ref[...]
        h = None
        for r in range(R):
            Sr = S[r]
            rc = rc_refs[r][...].reshape(B, CW)
            Wl = W[3 * r][...]
            o = jnp.broadcast_to(W[3 * r + 1][...], (B, 128))
            for c in range(nch):
                mean_c = (Sr[0, c] + Sr[1, c]).reshape(B, CW) * rc
                o = o + _dot_t(mean_c, Wl[:, CW * c:CW * (c + 1)])
            o = o + _dot_t(h1v, W[3 * r + 2][...])
            h = o if h is None else h + o
        if R > 1:
            h = h / float(R)
        out_ref[...] = (h1v + h) * 0.5

    grid = (n // B,)
    in_specs = (
        [pl.BlockSpec((NC, nch, B8, 8, CW), lambda i: (0, 0, i, 0, 0))
         for _ in range(R)]
        + [pl.BlockSpec((B8, 8, CW), lambda i: (i, 0, 0))
           for _ in range(R)]
        + [pl.BlockSpec((B, 128), lambda i: (i, 0))]
        + [spec for _ in range(R) for spec in (
            pl.BlockSpec((128, 128), lambda i: (0, 0)),
            pl.BlockSpec((1, 128), lambda i: (0, 0)),
            pl.BlockSpec((128, 128), lambda i: (0, 0)))]
    )
    args = list(Ss) + list(rcs) + [h1] + [w for ws in Ws for w in ws]
    return pl.pallas_call(
        body, grid=grid, in_specs=in_specs,
        out_specs=pl.BlockSpec((B, 128), lambda i: (i, 0)),
        out_shape=jax.ShapeDtypeStruct((n, 128), jnp.float32))(*args)


def kernel(x_user, x_recipe, x_ing, params, edge_rates, edge_rev_rates,
           edge_has, edge_rev_has):
    p = params
    n_user, d_in = x_user.shape
    n_recipe = x_recipe.shape[0]
    n_ing = x_ing.shape[0]
    nch1 = d_in // CW

    def chunks(x):
        return [x[:, CW * c:CW * (c + 1)] for c in range(x.shape[1] // CW)]

    def wset(pref, et):
        return (p[pref + "_" + et + "_Wl"],
                p[pref + "_" + et + "_b"].reshape(1, 128),
                p[pref + "_" + et + "_Wr"])

    xu_c, xr_c, xi_c = chunks(x_user), chunks(x_recipe), chunks(x_ing)

    S1 = _sc_segment_sums(
        [xu_c, xr_c, xr_c, xi_c],
        [edge_rates, edge_rev_rates, edge_has, edge_rev_has],
        [n_recipe, n_user, n_ing, n_recipe],
        with_counts=True)
    S1_ra, S1_rr, S1_ha, S1_rh = [
        s.reshape(NC, nch1 + 1, s.shape[2] // 8, 8, CW) for s in S1]

    h1r, h1r_c, (rc_ra, rc_rh) = _tc_layer1(
        n_recipe, d_in, [S1_ra, S1_rh], x_recipe,
        [wset("l1", "rates"), wset("l1", "rev_has")])
    h1u, h1u_c, (rc_rr,) = _tc_layer1(
        n_user, d_in, [S1_rr], x_user, [wset("l1", "rev_rates")])
    h1i, h1i_c, (rc_ha,) = _tc_layer1(
        n_ing, d_in, [S1_ha], x_ing, [wset("l1", "has")])

    def unfold(cs):
        return [c.reshape(-1, CW) for c in cs]

    S2 = _sc_segment_sums(
        [unfold(h1u_c), unfold(h1r_c), unfold(h1r_c), unfold(h1i_c)],
        [edge_rates, edge_rev_rates, edge_has, edge_rev_has],
        [n_recipe, n_user, n_ing, n_recipe],
        with_counts=False)
    S2_ra, S2_rr, S2_ha, S2_rh = [
        s.reshape(NC, 4, s.shape[2] // 8, 8, CW) for s in S2]

    hr = _tc_layer2(n_recipe, [S2_ra, S2_rh], [rc_ra, rc_rh], h1r,
                    [wset("l2", "rates"), wset("l2", "rev_has")])
    hu = _tc_layer2(n_user, [S2_rr], [rc_rr], h1u,
                    [wset("l2", "rev_rates")])
    hi = _tc_layer2(n_ing, [S2_ha], [rc_ha], h1i, [wset("l2", "has")])
    return hu, hr, hi


# speedup vs baseline: 4.9097x; 1.3847x over previous
import functools

import jax
import jax.numpy as jnp
from jax import lax
from jax.experimental import pallas as pl
from jax.experimental.pallas import tpu as pltpu
from jax.experimental.pallas import tpu_sc as plsc

NC = 2
NS = 16
NW = NC * NS
K = 128
CW = 32
ZR = 125


def _sc_segment_sums(tables_by_type, edges, n_dsts, with_counts):
    ntypes = len(edges)
    E = edges[0].shape[1]
    nj = -(-E // (NW * K))
    ep = NW * K * nj
    nchs = [len(t) for t in tables_by_type]

    srcs, dsts = [], []
    for ei, nd in zip(edges, n_dsts):
        pad = ep - E
        srcs.append(jnp.concatenate(
            [ei[0], jnp.zeros((pad,), jnp.int32)]).reshape(ep // K, K))
        dsts.append(jnp.concatenate(
            [ei[1], jnp.full((pad,), nd, jnp.int32)]).reshape(ep // K, K))

    out_types = []
    for t in range(ntypes):
        cch = nchs[t] + (1 if with_counts else 0)
        out_types.append(
            jax.ShapeDtypeStruct((NC, cch, n_dsts[t], CW), jnp.float32))

    mesh = plsc.VectorSubcoreMesh(core_axis_name="c", subcore_axis_name="s")

    WB = 14
    nwin = nj // WB
    assert nj == WB * nwin and nwin % 2 == 0

    @functools.partial(
        pl.kernel,
        out_type=tuple(out_types),
        mesh=mesh,
        compiler_params=pltpu.CompilerParams(use_tc_tiling_on_sc=False),
        scratch_types=[
            pltpu.VMEM((WB, K), jnp.int32),
            pltpu.VMEM((WB, K), jnp.int32),
            pltpu.VMEM((WB, K), jnp.int32),
            pltpu.VMEM((WB, K), jnp.int32),
            pltpu.VMEM((K, CW), jnp.float32),
            pltpu.VMEM((K, CW), jnp.float32),
            pltpu.VMEM((K, CW), jnp.float32),
            pltpu.VMEM((K, CW), jnp.float32),
            pltpu.VMEM((ZR, CW), jnp.float32),
            pltpu.VMEM_SHARED((50016, CW), jnp.float32),
            pltpu.SemaphoreType.DMA,
            pltpu.SemaphoreType.DMA,
            pltpu.SemaphoreType.DMA,
        ],
    )
    def sc_kernel(*refs):
        nt = sum(nchs)
        nin = nt + 2 * ntypes
        table_refs = []
        pos = 0
        for t in range(ntypes):
            table_refs.append(refs[pos:pos + nchs[t]])
            pos += nchs[t]
        src_refs = refs[nt:nt + ntypes]
        dst_refs = refs[nt + ntypes:nin]
        out_refs = refs[nin:nin + ntypes]
        (wsA, wdA, wsB, wdB, rows0, rows1, rows2, rows3, zbuf, acc,
         gsem, ssem, wsem) = refs[nin + ntypes:]
        rows = (rows0, rows1, rows2, rows3)

        cid = lax.axis_index("c")
        sid = lax.axis_index("s")
        wid = sid * NC + cid

        def init_rows(i, _):
            for h in range(CW // 16):
                zbuf[i, pl.ds(16 * h, 16)] = jnp.zeros((16,), jnp.float32)
            return 0
        lax.fori_loop(0, ZR, init_rows, 0)

        def win_slice(ref_hbm, w):
            return ref_hbm.at[pl.ds(wid * nj + w * WB, WB)]

        def run_pass(t, table_ref, out_ref, ck, n_dst):
            rpt = n_dst // NS
            nz = rpt // ZR
            base = sid * rpt

            def zb(i, _):
                pltpu.sync_copy(zbuf, acc.at[pl.ds(base + i * ZR, ZR)])
                return 0
            lax.fori_loop(0, nz, zb, 0)
            if table_ref is None:
                def init_ones(i, _):
                    for h in range(CW // 16):
                        rows0[i, pl.ds(16 * h, 16)] = jnp.ones(
                            (16,), jnp.float32)
                    return 0
                lax.fori_loop(0, K, init_ones, 0)
            plsc.subcore_barrier()

            if table_ref is not None:
                def process(ws, wd):
                    for b in range(min(3, WB)):
                        pltpu.async_copy(
                            table_ref.at[ws.at[b]], rows[b % 4], gsem)
                    for b in range(WB):
                        pltpu.make_async_copy(
                            table_ref.at[ws.at[b]], rows[b % 4],
                            gsem).wait()
                        if b + 3 < WB:
                            pltpu.async_copy(
                                table_ref.at[ws.at[b + 3]],
                                rows[(b + 3) % 4], gsem)
                        pltpu.sync_copy(
                            rows[b % 4], acc.at[wd.at[b]], add=True)
            else:
                def process(ws, wd):
                    for h in range(2):
                        for b in range(WB // 2):
                            pltpu.async_copy(
                                rows0, acc.at[wd.at[h * (WB // 2) + b]],
                                ssem, add=True)
                        for b in range(WB // 2):
                            pltpu.make_async_copy(
                                rows0, acc.at[wd.at[h * (WB // 2) + b]],
                                ssem).wait()

            def start_win(w, ws, wd):
                pltpu.async_copy(win_slice(src_refs[t], w), ws, wsem)
                pltpu.async_copy(win_slice(dst_refs[t], w), wd, wsem)

            def wait_win(w, ws, wd):
                pltpu.make_async_copy(
                    win_slice(src_refs[t], w), ws, wsem).wait()
                pltpu.make_async_copy(
                    win_slice(dst_refs[t], w), wd, wsem).wait()

            pltpu.sync_copy(win_slice(src_refs[t], 0), wsA)
            pltpu.sync_copy(win_slice(dst_refs[t], 0), wdA)
            start_win(1, wsB, wdB)

            def ww_body(ww, _):
                w0 = 2 * ww
                @pl.when(ww > 0)
                def _():
                    wait_win(w0, wsA, wdA)
                process(wsA, wdA)
                wait_win(w0 + 1, wsB, wdB)
                @pl.when(w0 + 2 < nwin)
                def _():
                    start_win(w0 + 2, wsA, wdA)
                process(wsB, wdB)
                @pl.when(w0 + 3 < nwin)
                def _():
                    start_win(w0 + 3, wsB, wdB)
                return 0
            lax.fori_loop(0, nwin // 2, ww_body, 0)
            plsc.subcore_barrier()

            def db(i, _):
                sl = pl.ds(base + i * ZR, ZR)
                pltpu.sync_copy(acc.at[sl], out_ref.at[cid, ck, sl])
                return 0
            lax.fori_loop(0, nz, db, 0)

        for t in range(ntypes):
            for c in range(nchs[t]):
                run_pass(t, table_refs[t][c], out_refs[t], c, n_dsts[t])
            if with_counts:
                run_pass(t, None, out_refs[t], nchs[t], n_dsts[t])

    flat_in = ([tb for t in tables_by_type for tb in t] + srcs + dsts)
    return sc_kernel(*flat_in)


def _dot_t(a, w):
    return lax.dot_general(
        a, w, (((1,), (1,)), ((), ())),
        precision=lax.Precision.HIGHEST,
        preferred_element_type=jnp.float32)


def _tc_layer1(n, d_in, Ss, x, Ws, B=2000):
    R = len(Ss)
    nch = d_in // CW
    B8 = B // 8

    def body(*refs):
        S = refs[:R]
        x_ref = refs[R]
        W = refs[R + 1:R + 1 + 3 * R]
        out = refs[R + 1 + 3 * R:]
        h_ref = out[0]
        hc_refs = out[1:5]
        rc_refs = out[5:5 + R]

        h = None
        for r in range(R):
            Sr = S[r]
            cnt = (Sr[0, nch] + Sr[1, nch]).reshape(B, CW)
            rc = 1.0 / jnp.maximum(cnt, 1.0)
            rc_refs[r][...] = rc.reshape(B8, 8, CW)
            Wl = W[3 * r][...]
            o = jnp.broadcast_to(W[3 * r + 1][...], (B, 128))
            for c in range(nch):
                mean_c = (Sr[0, c] + Sr[1, c]).reshape(B, CW) * rc
                o = o + _dot_t(mean_c, Wl[:, CW * c:CW * (c + 1)])
            o = o + _dot_t(x_ref[...], W[3 * r + 2][...])
            h = o if h is None else h + o
        if R > 1:
            h = h / float(R)
        h = jnp.maximum(h, 0.0)
        h_ref[...] = h
        for c in range(4):
            hc_refs[c][...] = h[:, CW * c:CW * (c + 1)].reshape(B8, 8, CW)

    grid = (n // B,)
    in_specs = (
        [pl.BlockSpec((NC, nch + 1, B8, 8, CW),
                      lambda i: (0, 0, i, 0, 0)) for _ in range(R)]
        + [pl.BlockSpec((B, d_in), lambda i: (i, 0))]
        + [spec for _ in range(R) for spec in (
            pl.BlockSpec((128, d_in), lambda i: (0, 0)),
            pl.BlockSpec((1, 128), lambda i: (0, 0)),
            pl.BlockSpec((128, d_in), lambda i: (0, 0)))]
    )
    out_shapes = ([jax.ShapeDtypeStruct((n, 128), jnp.float32)]
                  + [jax.ShapeDtypeStruct((n // 8, 8, CW), jnp.float32)
                     for _ in range(4 + R)])
    out_specs = ([pl.BlockSpec((B, 128), lambda i: (i, 0))]
                 + [pl.BlockSpec((B8, 8, CW), lambda i: (i, 0, 0))
                    for _ in range(4 + R)])
    args = list(Ss) + [x] + [w for ws in Ws for w in ws]
    outs = pl.pallas_call(
        body, grid=grid, in_specs=in_specs, out_specs=out_specs,
        out_shape=out_shapes)(*args)
    return outs[0], list(outs[1:5]), list(outs[5:5 + R])


def _tc_layer2(n, Ss, rcs, h1, Ws, B=2000):
    R = len(Ss)
    nch = 128 // CW
    B8 = B // 8

    def body(*refs):
        S = refs[:R]
        rc_refs = refs[R:2 * R]
        h1_ref = refs[2 * R]
        W = refs[2 * R + 1:2 * R + 1 + 3 * R]
        out_ref = refs[-1]

        h1v = h1_ref[...]
        h = None
        for r in range(R):
            Sr = S[r]
            rc = rc_refs[r][...].reshape(B, CW)
            Wl = W[3 * r][...]
            o = jnp.broadcast_to(W[3 * r + 1][...], (B, 128))
            for c in range(nch):
                mean_c = (Sr[0, c] + Sr[1, c]).reshape(B, CW) * rc
                o = o + _dot_t(mean_c, Wl[:, CW * c:CW * (c + 1)])
            o = o + _dot_t(h1v, W[3 * r + 2][...])
            h = o if h is None else h + o
        if R > 1:
            h = h / float(R)
        out_ref[...] = (h1v + h) * 0.5

    grid = (n // B,)
    in_specs = (
        [pl.BlockSpec((NC, nch, B8, 8, CW), lambda i: (0, 0, i, 0, 0))
         for _ in range(R)]
        + [pl.BlockSpec((B8, 8, CW), lambda i: (i, 0, 0))
           for _ in range(R)]
        + [pl.BlockSpec((B, 128), lambda i: (i, 0))]
        + [spec for _ in range(R) for spec in (
            pl.BlockSpec((128, 128), lambda i: (0, 0)),
            pl.BlockSpec((1, 128), lambda i: (0, 0)),
            pl.BlockSpec((128, 128), lambda i: (0, 0)))]
    )
    args = list(Ss) + list(rcs) + [h1] + [w for ws in Ws for w in ws]
    return pl.pallas_call(
        body, grid=grid, in_specs=in_specs,
        out_specs=pl.BlockSpec((B, 128), lambda i: (i, 0)),
        out_shape=jax.ShapeDtypeStruct((n, 128), jnp.float32))(*args)


def kernel(x_user, x_recipe, x_ing, params, edge_rates, edge_rev_rates,
           edge_has, edge_rev_has):
    p = params
    n_user, d_in = x_user.shape
    n_recipe = x_recipe.shape[0]
    n_ing = x_ing.shape[0]
    nch1 = d_in // CW

    def chunks(x):
        return [x[:, CW * c:CW * (c + 1)] for c in range(x.shape[1] // CW)]

    def wset(pref, et):
        return (p[pref + "_" + et + "_Wl"],
                p[pref + "_" + et + "_b"].reshape(1, 128),
                p[pref + "_" + et + "_Wr"])

    xu_c, xr_c, xi_c = chunks(x_user), chunks(x_recipe), chunks(x_ing)

    S1 = _sc_segment_sums(
        [xu_c, xr_c, xr_c, xi_c],
        [edge_rates, edge_rev_rates, edge_has, edge_rev_has],
        [n_recipe, n_user, n_ing, n_recipe],
        with_counts=True)
    S1_ra, S1_rr, S1_ha, S1_rh = [
        s.reshape(NC, nch1 + 1, s.shape[2] // 8, 8, CW) for s in S1]

    h1r, h1r_c, (rc_ra, rc_rh) = _tc_layer1(
        n_recipe, d_in, [S1_ra, S1_rh], x_recipe,
        [wset("l1", "rates"), wset("l1", "rev_has")])
    h1u, h1u_c, (rc_rr,) = _tc_layer1(
        n_user, d_in, [S1_rr], x_user, [wset("l1", "rev_rates")])
    h1i, h1i_c, (rc_ha,) = _tc_layer1(
        n_ing, d_in, [S1_ha], x_ing, [wset("l1", "has")])

    def unfold(cs):
        return [c.reshape(-1, CW) for c in cs]

    S2 = _sc_segment_sums(
        [unfold(h1u_c), unfold(h1r_c), unfold(h1r_c), unfold(h1i_c)],
        [edge_rates, edge_rev_rates, edge_has, edge_rev_has],
        [n_recipe, n_user, n_ing, n_recipe],
        with_counts=False)
    S2_ra, S2_rr, S2_ha, S2_rh = [
        s.reshape(NC, 4, s.shape[2] // 8, 8, CW) for s in S2]

    hr = _tc_layer2(n_recipe, [S2_ra, S2_rh], [rc_ra, rc_rh], h1r,
                    [wset("l2", "rates"), wset("l2", "rev_has")])
    hu = _tc_layer2(n_user, [S2_rr], [rc_rr], h1u,
                    [wset("l2", "rev_rates")])
    hi = _tc_layer2(n_ing, [S2_ha], [rc_ha], h1i, [wset("l2", "has")])
    return hu, hr, hi


# speedup vs baseline: 5.0009x; 1.0186x over previous
import functools

import jax
import jax.numpy as jnp
from jax import lax
from jax.experimental import pallas as pl
from jax.experimental.pallas import tpu as pltpu
from jax.experimental.pallas import tpu_sc as plsc

NC = 2
NS = 16
NW = NC * NS
K = 128
CW = 32
ZR = 125


def _sc_segment_sums(tables_by_type, edges, n_dsts, with_counts):
    ntypes = len(edges)
    E = edges[0].shape[1]
    nj = -(-E // (NW * K))
    ep = NW * K * nj
    nchs = [len(t) for t in tables_by_type]

    srcs, dsts = [], []
    for ei, nd in zip(edges, n_dsts):
        pad = ep - E
        srcs.append(jnp.concatenate(
            [ei[0], jnp.zeros((pad,), jnp.int32)]).reshape(ep // K, K))
        dsts.append(jnp.concatenate(
            [ei[1], nd + (jnp.arange(pad, dtype=jnp.int32) % 16)]
        ).reshape(ep // K, K))

    out_types = []
    for t in range(ntypes):
        cch = nchs[t] + (1 if with_counts else 0)
        out_types.append(
            jax.ShapeDtypeStruct((NC, cch, n_dsts[t], CW), jnp.float32))

    mesh = plsc.VectorSubcoreMesh(core_axis_name="c", subcore_axis_name="s")

    WB = 14
    nwin = nj // WB
    assert nj == WB * nwin and nwin % 2 == 0

    @functools.partial(
        pl.kernel,
        out_type=tuple(out_types),
        mesh=mesh,
        compiler_params=pltpu.CompilerParams(use_tc_tiling_on_sc=False),
        scratch_types=[
            pltpu.VMEM((WB, K), jnp.int32),
            pltpu.VMEM((WB, K), jnp.int32),
            pltpu.VMEM((WB, K), jnp.int32),
            pltpu.VMEM((WB, K), jnp.int32),
            pltpu.VMEM((K, CW), jnp.float32),
            pltpu.VMEM((K, CW), jnp.float32),
            pltpu.VMEM((K, CW), jnp.float32),
            pltpu.VMEM((K, CW), jnp.float32),
            pltpu.VMEM((ZR, CW), jnp.float32),
            pltpu.VMEM_SHARED((50016, CW), jnp.float32),
            pltpu.SemaphoreType.DMA,
            pltpu.SemaphoreType.DMA,
            pltpu.SemaphoreType.DMA,
        ],
    )
    def sc_kernel(*refs):
        nt = sum(nchs)
        nin = nt + 2 * ntypes
        table_refs = []
        pos = 0
        for t in range(ntypes):
            table_refs.append(refs[pos:pos + nchs[t]])
            pos += nchs[t]
        src_refs = refs[nt:nt + ntypes]
        dst_refs = refs[nt + ntypes:nin]
        out_refs = refs[nin:nin + ntypes]
        (wsA, wdA, wsB, wdB, rows0, rows1, rows2, rows3, zbuf, acc,
         gsem, ssem, wsem) = refs[nin + ntypes:]
        rows = (rows0, rows1, rows2, rows3)

        cid = lax.axis_index("c")
        sid = lax.axis_index("s")
        wid = sid * NC + cid

        def init_rows(i, _):
            for h in range(CW // 16):
                zbuf[i, pl.ds(16 * h, 16)] = jnp.zeros((16,), jnp.float32)
            return 0
        lax.fori_loop(0, ZR, init_rows, 0)

        def win_slice(ref_hbm, w):
            return ref_hbm.at[pl.ds(wid * nj + w * WB, WB)]

        def run_pass(t, table_ref, out_ref, ck, n_dst):
            rpt = n_dst // NS
            nz = rpt // ZR
            base = sid * rpt

            def zb(i, _):
                pltpu.sync_copy(zbuf, acc.at[pl.ds(base + i * ZR, ZR)])
                return 0
            lax.fori_loop(0, nz, zb, 0)
            if table_ref is None:
                def init_ones(i, _):
                    for h in range(CW // 16):
                        rows0[i, pl.ds(16 * h, 16)] = jnp.ones(
                            (16,), jnp.float32)
                    return 0
                lax.fori_loop(0, K, init_ones, 0)
            plsc.subcore_barrier()

            if table_ref is not None:
                def process(ws, wd):
                    for b in range(min(3, WB)):
                        pltpu.async_copy(
                            table_ref.at[ws.at[b]], rows[b % 4], gsem)
                    for b in range(WB):
                        pltpu.make_async_copy(
                            table_ref.at[ws.at[b]], rows[b % 4],
                            gsem).wait()
                        if b + 3 < WB:
                            pltpu.async_copy(
                                table_ref.at[ws.at[b + 3]],
                                rows[(b + 3) % 4], gsem)
                        pltpu.sync_copy(
                            rows[b % 4], acc.at[wd.at[b]], add=True)
            else:
                def process(ws, wd):
                    for h in range(2):
                        for b in range(WB // 2):
                            pltpu.async_copy(
                                rows0, acc.at[wd.at[h * (WB // 2) + b]],
                                ssem, add=True)
                        for b in range(WB // 2):
                            pltpu.make_async_copy(
                                rows0, acc.at[wd.at[h * (WB // 2) + b]],
                                ssem).wait()

            def start_win(w, ws, wd):
                pltpu.async_copy(win_slice(src_refs[t], w), ws, wsem)
                pltpu.async_copy(win_slice(dst_refs[t], w), wd, wsem)

            def wait_win(w, ws, wd):
                pltpu.make_async_copy(
                    win_slice(src_refs[t], w), ws, wsem).wait()
                pltpu.make_async_copy(
                    win_slice(dst_refs[t], w), wd, wsem).wait()

            pltpu.sync_copy(win_slice(src_refs[t], 0), wsA)
            pltpu.sync_copy(win_slice(dst_refs[t], 0), wdA)
            start_win(1, wsB, wdB)

            def ww_body(ww, _):
                w0 = 2 * ww
                @pl.when(ww > 0)
                def _():
                    wait_win(w0, wsA, wdA)
                process(wsA, wdA)
                wait_win(w0 + 1, wsB, wdB)
                @pl.when(w0 + 2 < nwin)
                def _():
                    start_win(w0 + 2, wsA, wdA)
                process(wsB, wdB)
                @pl.when(w0 + 3 < nwin)
                def _():
                    start_win(w0 + 3, wsB, wdB)
                return 0
            lax.fori_loop(0, nwin // 2, ww_body, 0)
            plsc.subcore_barrier()

            def db(i, _):
                sl = pl.ds(base + i * ZR, ZR)
                pltpu.sync_copy(acc.at[sl], out_ref.at[cid, ck, sl])
                return 0
            lax.fori_loop(0, nz, db, 0)

        for t in range(ntypes):
            for c in range(nchs[t]):
                run_pass(t, table_refs[t][c], out_refs[t], c, n_dsts[t])
            if with_counts:
                run_pass(t, None, out_refs[t], nchs[t], n_dsts[t])

    flat_in = ([tb for t in tables_by_type for tb in t] + srcs + dsts)
    return sc_kernel(*flat_in)


def _dot_t(a, w):
    return lax.dot_general(
        a, w, (((1,), (1,)), ((), ())),
        precision=lax.Precision.HIGHEST,
        preferred_element_type=jnp.float32)


def _tc_layer1(n, d_in, Ss, x, Ws, B=2000):
    R = len(Ss)
    nch = d_in // CW
    B8 = B // 8

    def body(*refs):
        S = refs[:R]
        x_ref = refs[R]
        W = refs[R + 1:R + 1 + 3 * R]
        out = refs[R + 1 + 3 * R:]
        h_ref = out[0]
        hc_refs = out[1:5]
        rc_refs = out[5:5 + R]

        h = None
        for r in range(R):
            Sr = S[r]
            cnt = (Sr[0, nch] + Sr[1, nch]).reshape(B, CW)
            rc = 1.0 / jnp.maximum(cnt, 1.0)
            rc_refs[r][...] = rc.reshape(B8, 8, CW)
            Wl = W[3 * r][...]
            o = jnp.broadcast_to(W[3 * r + 1][...], (B, 128))
            for c in range(nch):
                mean_c = (Sr[0, c] + Sr[1, c]).reshape(B, CW) * rc
                o = o + _dot_t(mean_c, Wl[:, CW * c:CW * (c + 1)])
            o = o + _dot_t(x_ref[...], W[3 * r + 2][...])
            h = o if h is None else h + o
        if R > 1:
            h = h / float(R)
        h = jnp.maximum(h, 0.0)
        h_ref[...] = h
        for c in range(4):
            hc_refs[c][...] = h[:, CW * c:CW * (c + 1)].reshape(B8, 8, CW)

    grid = (n // B,)
    in_specs = (
        [pl.BlockSpec((NC, nch + 1, B8, 8, CW),
                      lambda i: (0, 0, i, 0, 0)) for _ in range(R)]
        + [pl.BlockSpec((B, d_in), lambda i: (i, 0))]
        + [spec for _ in range(R) for spec in (
            pl.BlockSpec((128, d_in), lambda i: (0, 0)),
            pl.BlockSpec((1, 128), lambda i: (0, 0)),
            pl.BlockSpec((128, d_in), lambda i: (0, 0)))]
    )
    out_shapes = ([jax.ShapeDtypeStruct((n, 128), jnp.float32)]
                  + [jax.ShapeDtypeStruct((n // 8, 8, CW), jnp.float32)
                     for _ in range(4 + R)])
    out_specs = ([pl.BlockSpec((B, 128), lambda i: (i, 0))]
                 + [pl.BlockSpec((B8, 8, CW), lambda i: (i, 0, 0))
                    for _ in range(4 + R)])
    args = list(Ss) + [x] + [w for ws in Ws for w in ws]
    outs = pl.pallas_call(
        body, grid=grid, in_specs=in_specs, out_specs=out_specs,
        out_shape=out_shapes)(*args)
    return outs[0], list(outs[1:5]), list(outs[5:5 + R])


def _tc_layer2(n, Ss, rcs, h1, Ws, B=2000):
    R = len(Ss)
    nch = 128 // CW
    B8 = B // 8

    def body(*refs):
        S = refs[:R]
        rc_refs = refs[R:2 * R]
        h1_ref = refs[2 * R]
        W = refs[2 * R + 1:2 * R + 1 + 3 * R]
        out_ref = refs[-1]

        h1v = h1_ref[...]
        h = None
        for r in range(R):
            Sr = S[r]
            rc = rc_refs[r][...].reshape(B, CW)
            Wl = W[3 * r][...]
            o = jnp.broadcast_to(W[3 * r + 1][...], (B, 128))
            for c in range(nch):
                mean_c = (Sr[0, c] + Sr[1, c]).reshape(B, CW) * rc
                o = o + _dot_t(mean_c, Wl[:, CW * c:CW * (c + 1)])
            o = o + _dot_t(h1v, W[3 * r + 2][...])
            h = o if h is None else h + o
        if R > 1:
            h = h / float(R)
        out_ref[...] = (h1v + h) * 0.5

    grid = (n // B,)
    in_specs = (
        [pl.BlockSpec((NC, nch, B8, 8, CW), lambda i: (0, 0, i, 0, 0))
         for _ in range(R)]
        + [pl.BlockSpec((B8, 8, CW), lambda i: (i, 0, 0))
           for _ in range(R)]
        + [pl.BlockSpec((B, 128), lambda i: (i, 0))]
        + [spec for _ in range(R) for spec in (
            pl.BlockSpec((128, 128), lambda i: (0, 0)),
            pl.BlockSpec((1, 128), lambda i: (0, 0)),
            pl.BlockSpec((128, 128), lambda i: (0, 0)))]
    )
    args = list(Ss) + list(rcs) + [h1] + [w for ws in Ws for w in ws]
    return pl.pallas_call(
        body, grid=grid, in_specs=in_specs,
        out_specs=pl.BlockSpec((B, 128), lambda i: (i, 0)),
        out_shape=jax.ShapeDtypeStruct((n, 128), jnp.float32))(*args)


def kernel(x_user, x_recipe, x_ing, params, edge_rates, edge_rev_rates,
           edge_has, edge_rev_has):
    p = params
    n_user, d_in = x_user.shape
    n_recipe = x_recipe.shape[0]
    n_ing = x_ing.shape[0]
    nch1 = d_in // CW

    def chunks(x):
        return [x[:, CW * c:CW * (c + 1)] for c in range(x.shape[1] // CW)]

    def wset(pref, et):
        return (p[pref + "_" + et + "_Wl"],
                p[pref + "_" + et + "_b"].reshape(1, 128),
                p[pref + "_" + et + "_Wr"])

    xu_c, xr_c, xi_c = chunks(x_user), chunks(x_recipe), chunks(x_ing)

    S1 = _sc_segment_sums(
        [xu_c, xr_c, xr_c, xi_c],
        [edge_rates, edge_rev_rates, edge_has, edge_rev_has],
        [n_recipe, n_user, n_ing, n_recipe],
        with_counts=True)
    S1_ra, S1_rr, S1_ha, S1_rh = [
        s.reshape(NC, nch1 + 1, s.shape[2] // 8, 8, CW) for s in S1]

    h1r, h1r_c, (rc_ra, rc_rh) = _tc_layer1(
        n_recipe, d_in, [S1_ra, S1_rh], x_recipe,
        [wset("l1", "rates"), wset("l1", "rev_has")])
    h1u, h1u_c, (rc_rr,) = _tc_layer1(
        n_user, d_in, [S1_rr], x_user, [wset("l1", "rev_rates")])
    h1i, h1i_c, (rc_ha,) = _tc_layer1(
        n_ing, d_in, [S1_ha], x_ing, [wset("l1", "has")])

    def unfold(cs):
        return [c.reshape(-1, CW) for c in cs]

    S2 = _sc_segment_sums(
        [unfold(h1u_c), unfold(h1r_c), unfold(h1r_c), unfold(h1i_c)],
        [edge_rates, edge_rev_rates, edge_has, edge_rev_has],
        [n_recipe, n_user, n_ing, n_recipe],
        with_counts=False)
    S2_ra, S2_rr, S2_ha, S2_rh = [
        s.reshape(NC, 4, s.shape[2] // 8, 8, CW) for s in S2]

    hr = _tc_layer2(n_recipe, [S2_ra, S2_rh], [rc_ra, rc_rh], h1r,
                    [wset("l2", "rates"), wset("l2", "rev_has")])
    hu = _tc_layer2(n_user, [S2_rr], [rc_rr], h1u,
                    [wset("l2", "rev_rates")])
    hi = _tc_layer2(n_ing, [S2_ha], [rc_ha], h1i, [wset("l2", "has")])
    return hu, hr, hi


# speedup vs baseline: 5.8143x; 1.1627x over previous
import functools

import jax
import jax.numpy as jnp
from jax import lax
from jax.experimental import pallas as pl
from jax.experimental.pallas import tpu as pltpu
from jax.experimental.pallas import tpu_sc as plsc

NC = 2
NS = 16
NW = NC * NS
K = 128
CW = 32
ZR = 125


def _sc_segment_sums(tables_by_type, edges, n_dsts, with_counts):
    ntypes = len(edges)
    E = edges[0].shape[1]
    nj = -(-E // (NW * K))
    ep = NW * K * nj
    nchs = [len(t) for t in tables_by_type]

    srcs, dsts = [], []
    for ei, nd in zip(edges, n_dsts):
        pad = ep - E
        srcs.append(jnp.concatenate(
            [ei[0], jnp.zeros((pad,), jnp.int32)]).reshape(ep // K, K))
        dsts.append(jnp.concatenate(
            [ei[1], nd + (jnp.arange(pad, dtype=jnp.int32) % 16)]
        ).reshape(ep // K, K))

    out_types = []
    for t in range(ntypes):
        cch = nchs[t] + (1 if with_counts else 0)
        out_types.append(
            jax.ShapeDtypeStruct((NC, cch, n_dsts[t], CW), jnp.float32))

    mesh = plsc.VectorSubcoreMesh(core_axis_name="c", subcore_axis_name="s")

    WB = 14
    nwin = nj // WB
    assert nj == WB * nwin and nwin % 2 == 0

    @functools.partial(
        pl.kernel,
        out_type=tuple(out_types),
        mesh=mesh,
        compiler_params=pltpu.CompilerParams(use_tc_tiling_on_sc=False),
        scratch_types=[
            pltpu.VMEM((WB, K), jnp.int32),
            pltpu.VMEM((WB, K), jnp.int32),
            pltpu.VMEM((WB, K), jnp.int32),
            pltpu.VMEM((WB, K), jnp.int32),
            pltpu.VMEM((K, CW), jnp.float32),
            pltpu.VMEM((K, CW), jnp.float32),
            pltpu.VMEM((K, CW), jnp.float32),
            pltpu.VMEM((K, CW), jnp.float32),
            pltpu.VMEM((ZR, CW), jnp.float32),
            pltpu.VMEM_SHARED((50016, CW), jnp.float32),
            pltpu.SemaphoreType.DMA,
            pltpu.SemaphoreType.DMA,
            pltpu.SemaphoreType.DMA,
        ],
    )
    def sc_kernel(*refs):
        nt = sum(nchs)
        nin = nt + 2 * ntypes
        table_refs = []
        pos = 0
        for t in range(ntypes):
            table_refs.append(refs[pos:pos + nchs[t]])
            pos += nchs[t]
        src_refs = refs[nt:nt + ntypes]
        dst_refs = refs[nt + ntypes:nin]
        out_refs = refs[nin:nin + ntypes]
        (wsA, wdA, wsB, wdB, rows0, rows1, rows2, rows3, zbuf, acc,
         gsem, ssem, wsem) = refs[nin + ntypes:]
        rows = (rows0, rows1, rows2, rows3)

        cid = lax.axis_index("c")
        sid = lax.axis_index("s")
        wid = sid * NC + cid

        def init_rows(i, _):
            for h in range(CW // 16):
                zbuf[i, pl.ds(16 * h, 16)] = jnp.zeros((16,), jnp.float32)
            return 0
        lax.fori_loop(0, ZR, init_rows, 0)

        def win_slice(ref_hbm, w):
            return ref_hbm.at[pl.ds(wid * nj + w * WB, WB)]

        def run_pass(t, table_ref, out_ref, ck, n_dst):
            rpt = n_dst // NS
            nz = rpt // ZR
            base = sid * rpt

            def zb(i, _):
                pltpu.sync_copy(zbuf, acc.at[pl.ds(base + i * ZR, ZR)])
                return 0
            lax.fori_loop(0, nz, zb, 0)
            if table_ref is None:
                def init_ones(i, _):
                    for h in range(CW // 16):
                        rows0[i, pl.ds(16 * h, 16)] = jnp.ones(
                            (16,), jnp.float32)
                    return 0
                lax.fori_loop(0, K, init_ones, 0)
            plsc.subcore_barrier()

            if table_ref is not None:
                def process(ws, wd):
                    for b in range(min(3, WB)):
                        pltpu.async_copy(
                            table_ref.at[ws.at[b]], rows[b % 4], gsem)
                    for b in range(WB):
                        pltpu.make_async_copy(
                            table_ref.at[ws.at[b]], rows[b % 4],
                            gsem).wait()
                        if b + 3 < WB:
                            pltpu.async_copy(
                                table_ref.at[ws.at[b + 3]],
                                rows[(b + 3) % 4], gsem)
                        pltpu.sync_copy(
                            rows[b % 4], acc.at[wd.at[b]], add=True)
            else:
                def process(ws, wd):
                    for h in range(2):
                        for b in range(WB // 2):
                            pltpu.async_copy(
                                rows0, acc.at[wd.at[h * (WB // 2) + b]],
                                ssem, add=True)
                        for b in range(WB // 2):
                            pltpu.make_async_copy(
                                rows0, acc.at[wd.at[h * (WB // 2) + b]],
                                ssem).wait()

            def start_win(w, ws, wd):
                pltpu.async_copy(win_slice(src_refs[t], w), ws, wsem)
                pltpu.async_copy(win_slice(dst_refs[t], w), wd, wsem)

            def wait_win(w, ws, wd):
                pltpu.make_async_copy(
                    win_slice(src_refs[t], w), ws, wsem).wait()
                pltpu.make_async_copy(
                    win_slice(dst_refs[t], w), wd, wsem).wait()

            pltpu.sync_copy(win_slice(src_refs[t], 0), wsA)
            pltpu.sync_copy(win_slice(dst_refs[t], 0), wdA)
            start_win(1, wsB, wdB)

            def ww_body(ww, _):
                w0 = 2 * ww
                @pl.when(ww > 0)
                def _():
                    wait_win(w0, wsA, wdA)
                process(wsA, wdA)
                wait_win(w0 + 1, wsB, wdB)
                @pl.when(w0 + 2 < nwin)
                def _():
                    start_win(w0 + 2, wsA, wdA)
                process(wsB, wdB)
                @pl.when(w0 + 3 < nwin)
                def _():
                    start_win(w0 + 3, wsB, wdB)
                return 0
            lax.fori_loop(0, nwin // 2, ww_body, 0)
            plsc.subcore_barrier()

            def db(i, _):
                sl = pl.ds(base + i * ZR, ZR)
                pltpu.sync_copy(acc.at[sl], out_ref.at[cid, ck, sl])
                return 0
            lax.fori_loop(0, nz, db, 0)

        for t in range(ntypes):
            for c in range(nchs[t]):
                run_pass(t, table_refs[t][c], out_refs[t], c, n_dsts[t])
            if with_counts:
                run_pass(t, None, out_refs[t], nchs[t], n_dsts[t])

    flat_in = ([tb for t in tables_by_type for tb in t] + srcs + dsts)
    return sc_kernel(*flat_in)


def _dot_t(a, w):
    return lax.dot_general(
        a, w, (((1,), (1,)), ((), ())),
        precision=lax.Precision.HIGHEST,
        preferred_element_type=jnp.float32)


def _tc_layer1(n, d_in, Ss, x, Ws, B=2000):
    R = len(Ss)
    nch = d_in // CW
    B8 = B // 8

    def body(*refs):
        S = refs[:R]
        x_ref = refs[R]
        W = refs[R + 1:R + 1 + 3 * R]
        out = refs[R + 1 + 3 * R:]
        h_ref = out[0]
        hc_refs = out[1:5]
        rc_refs = out[5:5 + R]

        h = None
        for r in range(R):
            Sr = S[r]
            cnt = (Sr[0, nch] + Sr[1, nch]).reshape(B, CW)
            rc = 1.0 / jnp.maximum(cnt, 1.0)
            rc_refs[r][...] = rc.reshape(B8, 8, CW)
            Wl = W[3 * r][...]
            o = jnp.broadcast_to(W[3 * r + 1][...], (B, 128))
            for c in range(nch):
                mean_c = (Sr[0, c] + Sr[1, c]).reshape(B, CW) * rc
                o = o + _dot_t(mean_c, Wl[:, CW * c:CW * (c + 1)])
            o = o + _dot_t(x_ref[...], W[3 * r + 2][...])
            h = o if h is None else h + o
        if R > 1:
            h = h / float(R)
        h = jnp.maximum(h, 0.0)
        h_ref[...] = h
        for c in range(4):
            hc_refs[c][...] = h[:, CW * c:CW * (c + 1)].reshape(B8, 8, CW)

    grid = (n // B,)
    in_specs = (
        [pl.BlockSpec((NC, nch + 1, B8, 8, CW),
                      lambda i: (0, 0, i, 0, 0)) for _ in range(R)]
        + [pl.BlockSpec((B, d_in), lambda i: (i, 0))]
        + [spec for _ in range(R) for spec in (
            pl.BlockSpec((128, d_in), lambda i: (0, 0)),
            pl.BlockSpec((1, 128), lambda i: (0, 0)),
            pl.BlockSpec((128, d_in), lambda i: (0, 0)))]
    )
    out_shapes = ([jax.ShapeDtypeStruct((n, 128), jnp.float32)]
                  + [jax.ShapeDtypeStruct((n // 8, 8, CW), jnp.float32)
                     for _ in range(4 + R)])
    out_specs = ([pl.BlockSpec((B, 128), lambda i: (i, 0))]
                 + [pl.BlockSpec((B8, 8, CW), lambda i: (i, 0, 0))
                    for _ in range(4 + R)])
    args = list(Ss) + [x] + [w for ws in Ws for w in ws]
    outs = pl.pallas_call(
        body, grid=grid, in_specs=in_specs, out_specs=out_specs,
        out_shape=out_shapes)(*args)
    return outs[0], list(outs[1:5]), list(outs[5:5 + R])


def _tc_layer2(n, Ss, rcs, h1, Ws, B=2000):
    R = len(Ss)
    nch = 128 // CW
    B8 = B // 8

    def body(*refs):
        S = refs[:R]
        rc_refs = refs[R:2 * R]
        h1_ref = refs[2 * R]
        W = refs[2 * R + 1:2 * R + 1 + 3 * R]
        out_ref = refs[-1]

        h1v = h1_ref[...]
        h = None
        for r in range(R):
            Sr = S[r]
            rc = rc_refs[r][...].reshape(B, CW)
            Wl = W[3 * r][...]
            o = jnp.broadcast_to(W[3 * r + 1][...], (B, 128))
            for c in range(nch):
                mean_c = (Sr[0, c] + Sr[1, c]).reshape(B, CW) * rc
                o = o + _dot_t(mean_c, Wl[:, CW * c:CW * (c + 1)])
            o = o + _dot_t(h1v, W[3 * r + 2][...])
            h = o if h is None else h + o
        if R > 1:
            h = h / float(R)
        out_ref[...] = (h1v + h) * 0.5

    grid = (n // B,)
    in_specs = (
        [pl.BlockSpec((NC, nch, B8, 8, CW), lambda i: (0, 0, i, 0, 0))
         for _ in range(R)]
        + [pl.BlockSpec((B8, 8, CW), lambda i: (i, 0, 0))
           for _ in range(R)]
        + [pl.BlockSpec((B, 128), lambda i: (i, 0))]
        + [spec for _ in range(R) for spec in (
            pl.BlockSpec((128, 128), lambda i: (0, 0)),
            pl.BlockSpec((1, 128), lambda i: (0, 0)),
            pl.BlockSpec((128, 128), lambda i: (0, 0)))]
    )
    args = list(Ss) + list(rcs) + [h1] + [w for ws in Ws for w in ws]
    return pl.pallas_call(
        body, grid=grid, in_specs=in_specs,
        out_specs=pl.BlockSpec((B, 128), lambda i: (i, 0)),
        out_shape=jax.ShapeDtypeStruct((n, 128), jnp.float32))(*args)


def kernel(x_user, x_recipe, x_ing, params, edge_rates, edge_rev_rates,
           edge_has, edge_rev_has):
    p = params
    n_user, d_in = x_user.shape
    n_recipe = x_recipe.shape[0]
    n_ing = x_ing.shape[0]
    nch1 = d_in // CW

    def chunks(x):
        return [x[:, CW * c:CW * (c + 1)] for c in range(x.shape[1] // CW)]

    def wset(pref, et):
        return (p[pref + "_" + et + "_Wl"],
                p[pref + "_" + et + "_b"].reshape(1, 128),
                p[pref + "_" + et + "_Wr"])

    xu_c, xr_c, xi_c = chunks(x_user), chunks(x_recipe), chunks(x_ing)

    def sc1(tabs, ei, nd):
        (s,) = _sc_segment_sums([tabs], [ei], [nd], with_counts=True)
        return s.reshape(NC, nch1 + 1, nd // 8, 8, CW)

    S1_rr = sc1(xr_c, edge_rev_rates, n_user)
    S1_ha = sc1(xr_c, edge_has, n_ing)
    S1_ra = sc1(xu_c, edge_rates, n_recipe)
    S1_rh = sc1(xi_c, edge_rev_has, n_recipe)

    h1r, h1r_c, (rc_ra, rc_rh) = _tc_layer1(
        n_recipe, d_in, [S1_ra, S1_rh], x_recipe,
        [wset("l1", "rates"), wset("l1", "rev_has")])
    h1u, h1u_c, (rc_rr,) = _tc_layer1(
        n_user, d_in, [S1_rr], x_user, [wset("l1", "rev_rates")])
    h1i, h1i_c, (rc_ha,) = _tc_layer1(
        n_ing, d_in, [S1_ha], x_ing, [wset("l1", "has")])

    def unfold(cs):
        return [c.reshape(-1, CW) for c in cs]

    def sc2(tabs, ei, nd):
        (s,) = _sc_segment_sums([tabs], [ei], [nd], with_counts=False)
        return s.reshape(NC, 4, nd // 8, 8, CW)

    S2_ra = sc2(unfold(h1u_c), edge_rates, n_recipe)
    S2_rh = sc2(unfold(h1i_c), edge_rev_has, n_recipe)
    S2_rr = sc2(unfold(h1r_c), edge_rev_rates, n_user)
    S2_ha = sc2(unfold(h1r_c), edge_has, n_ing)

    hr = _tc_layer2(n_recipe, [S2_ra, S2_rh], [rc_ra, rc_rh], h1r,
                    [wset("l2", "rates"), wset("l2", "rev_has")])
    hu = _tc_layer2(n_user, [S2_rr], [rc_rr], h1u,
                    [wset("l2", "rev_rates")])
    hi = _tc_layer2(n_ing, [S2_ha], [rc_ha], h1i, [wset("l2", "has")])
    return hu, hr, hi
